# baseline (speedup 1.0000x reference)
import functools

import jax
import jax.numpy as jnp
from jax import lax
from jax.experimental import pallas as pl
from jax.experimental.pallas import tpu as pltpu
from jax.experimental.pallas import tpu_sc as plsc

_L = 1000
_BT = 128
_PIECES = ((0, 128), (128, 128), (256, 128), (384, 128), (512, 128), (640, 128), (768, 128), (896, 104))
_XRING = 4
_NUM_CORES = 2
_NUM_SUBCORES = 16


@functools.partial(jax.jit, static_argnums=(1, 2))
def _one_hot_sc_t(xt, b, f):
    nw = _NUM_CORES * _NUM_SUBCORES
    nbt = b // _BT
    spw = f * nbt // nw
    mesh = plsc.VectorSubcoreMesh(core_axis_name="c", subcore_axis_name="s",
                                  num_cores=_NUM_CORES,
                                  num_subcores=_NUM_SUBCORES)

    @functools.partial(
        pl.kernel,
        out_type=jax.ShapeDtypeStruct((f, _L, b), jnp.float32),
        mesh=mesh,
        compiler_params=pltpu.CompilerParams(needs_layout_passes=False),
        scratch_types=[
            pltpu.VMEM((_XRING, _BT), jnp.int32),
            pltpu.VMEM((_L, _BT), jnp.float32),
            pltpu.SemaphoreType.DMA,
        ] + [pltpu.SemaphoreType.DMA] * len(_PIECES),
    )
    def k(xt_hbm, out_hbm, xbuf, slab, sem_x, *sems):
        wid = lax.axis_index("s") * _NUM_CORES + lax.axis_index("c")
        base_sid = wid * spw

        zero16 = jnp.zeros((16,), jnp.float32)
        one16 = jnp.ones((16,), jnp.float32)
        iota16 = lax.iota(jnp.int32, 16)

        def fetch_x(j):
            sid = base_sid + j
            pltpu.async_copy(
                xt_hbm.at[sid // nbt, pl.ds((sid % nbt) * _BT, _BT)],
                xbuf.at[j % _XRING],
                sem_x,
            )

        def drain_x():
            pltpu.make_async_copy(
                xt_hbm.at[0, pl.ds(0, _BT)], xbuf.at[0], sem_x
            ).wait()

        def drain_piece(h):
            l0, hlen = _PIECES[h]
            pltpu.make_async_copy(
                slab.at[pl.ds(l0, hlen)],
                out_hbm.at[0, pl.ds(l0, hlen), pl.ds(0, _BT)],
                sems[h],
            ).wait()

        def fire_piece(h, j):
            l0, hlen = _PIECES[h]
            sid = base_sid + j
            pltpu.async_copy(
                slab.at[pl.ds(l0, hlen)],
                out_hbm.at[
                    sid // nbt, pl.ds(l0, hlen), pl.ds((sid % nbt) * _BT, _BT)
                ],
                sems[h],
            )

        def touch(j, l0, hlen, val16):
            for c in range(_BT // 16):
                xv = xbuf[j % _XRING, pl.ds(c * 16, 16)]
                m = (xv >= l0) & (xv < l0 + hlen)
                plsc.store_scatter(slab, [xv, iota16 + c * 16], val16, mask=m)

        def zero_rows(i, carry):
            for r in range(8):
                for c in range(_BT // 16):
                    slab[i * 8 + r, pl.ds(c * 16, 16)] = zero16
            return carry

        for j in range(_XRING - 1):
            fetch_x(j)
        drain_x()
        for h, (l0, hlen) in enumerate(_PIECES):
            lax.fori_loop(l0 // 8, (l0 + hlen) // 8, zero_rows, 0)
            touch(0, l0, hlen, one16)
            fire_piece(h, 0)
        fetch_x(_XRING - 1)

        def body(j, carry):
            drain_x()
            for h, (l0, hlen) in enumerate(_PIECES):
                drain_piece(h)
                touch(j - 1, l0, hlen, zero16)
                touch(j, l0, hlen, one16)
                fire_piece(h, j)

            @pl.when(j + _XRING - 1 < spw)
            def _():
                fetch_x(j + _XRING - 1)

            return carry

        lax.fori_loop(1, spw, body, 0)
        for h in range(len(_PIECES)):
            drain_piece(h)

    return k(xt)


def kernel(x):
    b, f = x.shape
    out_t = _one_hot_sc_t(x.T, b, f)
    return out_t.transpose(2, 0, 1)


# speedup vs baseline: 1.0355x; 1.0355x over previous
import functools

import jax
import jax.numpy as jnp
from jax import lax
from jax.experimental import pallas as pl
from jax.experimental.pallas import tpu as pltpu
from jax.experimental.pallas import tpu_sc as plsc

_L = 1000
_BT = 128
_PIECES = ((0, 256), (256, 256), (512, 256), (768, 232))
_XRING = 4
_NUM_CORES = 2
_NUM_SUBCORES = 16


@functools.partial(jax.jit, static_argnums=(1, 2))
def _one_hot_sc_t(xt, b, f):
    nw = _NUM_CORES * _NUM_SUBCORES
    nbt = b // _BT
    spw = f * nbt // nw
    mesh = plsc.VectorSubcoreMesh(core_axis_name="c", subcore_axis_name="s",
                                  num_cores=_NUM_CORES,
                                  num_subcores=_NUM_SUBCORES)

    @functools.partial(
        pl.kernel,
        out_type=jax.ShapeDtypeStruct((f, _L, b), jnp.float32),
        mesh=mesh,
        compiler_params=pltpu.CompilerParams(needs_layout_passes=False),
        scratch_types=[
            pltpu.VMEM((_XRING, _BT), jnp.int32),
            pltpu.VMEM((_L, _BT), jnp.float32),
            pltpu.SemaphoreType.DMA,
        ] + [pltpu.SemaphoreType.DMA] * len(_PIECES),
    )
    def k(xt_hbm, out_hbm, xbuf, slab, sem_x, *sems):
        wid = lax.axis_index("s") * _NUM_CORES + lax.axis_index("c")

        zero16 = jnp.zeros((16,), jnp.float32)
        one16 = jnp.ones((16,), jnp.float32)
        iota16 = lax.iota(jnp.int32, 16)

        def fetch_x(j):
            sid = j * nw + wid
            pltpu.async_copy(
                xt_hbm.at[sid // nbt, pl.ds((sid % nbt) * _BT, _BT)],
                xbuf.at[j % _XRING],
                sem_x,
            )

        def drain_x():
            pltpu.make_async_copy(
                xt_hbm.at[0, pl.ds(0, _BT)], xbuf.at[0], sem_x
            ).wait()

        def drain_piece(h):
            l0, hlen = _PIECES[h]
            pltpu.make_async_copy(
                slab.at[pl.ds(l0, hlen)],
                out_hbm.at[0, pl.ds(l0, hlen), pl.ds(0, _BT)],
                sems[h],
            ).wait()

        def fire_piece(h, j):
            l0, hlen = _PIECES[h]
            sid = j * nw + wid
            pltpu.async_copy(
                slab.at[pl.ds(l0, hlen)],
                out_hbm.at[
                    sid // nbt, pl.ds(l0, hlen), pl.ds((sid % nbt) * _BT, _BT)
                ],
                sems[h],
            )

        def touch(j, l0, hlen, val16):
            for c in range(_BT // 16):
                xv = xbuf[j % _XRING, pl.ds(c * 16, 16)]
                m = (xv >= l0) & (xv < l0 + hlen)
                plsc.store_scatter(slab, [xv, iota16 + c * 16], val16, mask=m)

        def zero_rows(i, carry):
            for r in range(8):
                for c in range(_BT // 16):
                    slab[i * 8 + r, pl.ds(c * 16, 16)] = zero16
            return carry

        for j in range(_XRING - 1):
            fetch_x(j)
        drain_x()
        for h, (l0, hlen) in enumerate(_PIECES):
            lax.fori_loop(l0 // 8, (l0 + hlen) // 8, zero_rows, 0)
            touch(0, l0, hlen, one16)
            fire_piece(h, 0)
        fetch_x(_XRING - 1)

        def body(j, carry):
            drain_x()
            for h, (l0, hlen) in enumerate(_PIECES):
                drain_piece(h)
                touch(j - 1, l0, hlen, zero16)
                touch(j, l0, hlen, one16)
                fire_piece(h, j)

            @pl.when(j + _XRING - 1 < spw)
            def _():
                fetch_x(j + _XRING - 1)

            return carry

        lax.fori_loop(1, spw, body, 0)
        for h in range(len(_PIECES)):
            drain_piece(h)

    return k(xt)


def kernel(x):
    b, f = x.shape
    out_t = _one_hot_sc_t(x.T, b, f)
    return out_t.transpose(2, 0, 1)


# speedup vs baseline: 1.0507x; 1.0146x over previous
import functools

import jax
import jax.numpy as jnp
from jax import lax
from jax.experimental import pallas as pl
from jax.experimental.pallas import tpu as pltpu
from jax.experimental.pallas import tpu_sc as plsc

_L = 1000
_BT = 128
_PIECES = ((0, 256), (256, 256), (512, 256), (768, 232))
_XRING = 4
_NUM_CORES = 2
_NUM_SUBCORES = 16


@functools.partial(jax.jit, static_argnums=(1, 2))
def _one_hot_sc_t(xt, b, f):
    nw = _NUM_CORES * _NUM_SUBCORES
    nbt = b // _BT
    spw = f * nbt // nw
    mesh = plsc.VectorSubcoreMesh(core_axis_name="c", subcore_axis_name="s",
                                  num_cores=_NUM_CORES,
                                  num_subcores=_NUM_SUBCORES)

    @functools.partial(
        pl.kernel,
        out_type=jax.ShapeDtypeStruct((f, _L, b), jnp.float32),
        mesh=mesh,
        compiler_params=pltpu.CompilerParams(needs_layout_passes=False),
        scratch_types=[
            pltpu.VMEM((_XRING, _BT), jnp.int32),
            pltpu.VMEM((_L, _BT), jnp.float32),
            pltpu.SemaphoreType.DMA,
        ] + [pltpu.SemaphoreType.DMA] * len(_PIECES),
    )
    def k(xt_hbm, out_hbm, xbuf, slab, sem_x, *sems):
        wid = lax.axis_index("c") * _NUM_SUBCORES + lax.axis_index("s")

        zero16 = jnp.zeros((16,), jnp.float32)
        one16 = jnp.ones((16,), jnp.float32)
        iota16 = lax.iota(jnp.int32, 16)

        def fetch_x(j):
            sid = j * nw + wid
            pltpu.async_copy(
                xt_hbm.at[sid // nbt, pl.ds((sid % nbt) * _BT, _BT)],
                xbuf.at[j % _XRING],
                sem_x,
            )

        def drain_x():
            pltpu.make_async_copy(
                xt_hbm.at[0, pl.ds(0, _BT)], xbuf.at[0], sem_x
            ).wait()

        def drain_piece(h):
            l0, hlen = _PIECES[h]
            pltpu.make_async_copy(
                slab.at[pl.ds(l0, hlen)],
                out_hbm.at[0, pl.ds(l0, hlen), pl.ds(0, _BT)],
                sems[h],
            ).wait()

        def fire_piece(h, j):
            l0, hlen = _PIECES[h]
            sid = j * nw + wid
            pltpu.async_copy(
                slab.at[pl.ds(l0, hlen)],
                out_hbm.at[
                    sid // nbt, pl.ds(l0, hlen), pl.ds((sid % nbt) * _BT, _BT)
                ],
                sems[h],
            )

        def touch(j, l0, hlen, val16):
            for c in range(_BT // 16):
                xv = xbuf[j % _XRING, pl.ds(c * 16, 16)]
                m = (xv >= l0) & (xv < l0 + hlen)
                plsc.store_scatter(slab, [xv, iota16 + c * 16], val16, mask=m)

        def zero_rows(i, carry):
            for r in range(8):
                for c in range(_BT // 16):
                    slab[i * 8 + r, pl.ds(c * 16, 16)] = zero16
            return carry

        for j in range(_XRING - 1):
            fetch_x(j)
        drain_x()
        for h, (l0, hlen) in enumerate(_PIECES):
            lax.fori_loop(l0 // 8, (l0 + hlen) // 8, zero_rows, 0)
            touch(0, l0, hlen, one16)
            fire_piece(h, 0)
        fetch_x(_XRING - 1)

        def body(j, carry):
            drain_x()
            for h, (l0, hlen) in enumerate(_PIECES):
                drain_piece(h)
                touch(j - 1, l0, hlen, zero16)
                touch(j, l0, hlen, one16)
                fire_piece(h, j)

            @pl.when(j + _XRING - 1 < spw)
            def _():
                fetch_x(j + _XRING - 1)

            return carry

        lax.fori_loop(1, spw, body, 0)
        for h in range(len(_PIECES)):
            drain_piece(h)

    return k(xt)


def kernel(x):
    b, f = x.shape
    out_t = _one_hot_sc_t(x.T, b, f)
    return out_t.transpose(2, 0, 1)
